# initial kernel scaffold (unmeasured)
import jax
import jax.numpy as jnp
from jax import lax
from jax.experimental import pallas as pl
from jax.experimental.pallas import tpu as pltpu

N_DEV = 4


def kernel(A, B):
    A = A.astype(jnp.bfloat16)
    B = B.astype(jnp.bfloat16)
    M, K = A.shape
    _, N = B.shape
    CH = M // N_DEV

    def body(a_ref, b_ref, out_ref, comm_ref, send_sems, recv_sems):
        p = lax.axis_index("i")
        left = (p - 1) % N_DEV
        right = (p + 1) % N_DEV

        barrier_sem = pltpu.get_barrier_semaphore()
        for nbr in [left, right]:
            pl.semaphore_signal(
                barrier_sem, inc=1,
                device_id=(nbr,), device_id_type=pl.DeviceIdType.MESH,
            )
        pl.semaphore_wait(barrier_sem, 2)

        for c in range(N_DEV):
            out_ref[pl.ds(c * CH, CH), :] = jnp.dot(
                a_ref[pl.ds(c * CH, CH), :], b_ref[:, :],
                preferred_element_type=jnp.float32,
            )

        for s in range(N_DEV - 1):
            send_c = (p + (N_DEV - 1) - s) % N_DEV
            recv_c = (p + (N_DEV - 2) - s) % N_DEV
            rdma = pltpu.make_async_remote_copy(
                src_ref=out_ref.at[pl.ds(send_c * CH, CH), :],
                dst_ref=comm_ref.at[s],
                send_sem=send_sems.at[s],
                recv_sem=recv_sems.at[s],
                device_id=(right,),
                device_id_type=pl.DeviceIdType.MESH,
            )
            rdma.start()
            rdma.wait()
            out_ref[pl.ds(recv_c * CH, CH), :] = (
                out_ref[pl.ds(recv_c * CH, CH), :] + comm_ref[s]
            )

        for s in range(N_DEV - 1):
            send_c = (p - s) % N_DEV
            rdma = pltpu.make_async_remote_copy(
                src_ref=out_ref.at[pl.ds(send_c * CH, CH), :],
                dst_ref=out_ref.at[pl.ds(send_c * CH, CH), :],
                send_sem=send_sems.at[N_DEV - 1 + s],
                recv_sem=recv_sems.at[N_DEV - 1 + s],
                device_id=(right,),
                device_id_type=pl.DeviceIdType.MESH,
            )
            rdma.start()
            rdma.wait()

    return pl.pallas_call(
        body,
        out_shape=jax.ShapeDtypeStruct((M, N), jnp.float32),
        in_specs=[
            pl.BlockSpec(memory_space=pltpu.VMEM),
            pl.BlockSpec(memory_space=pltpu.VMEM),
        ],
        out_specs=pl.BlockSpec(memory_space=pltpu.VMEM),
        scratch_shapes=[
            pltpu.VMEM((N_DEV - 1, CH, N), jnp.float32),
            pltpu.SemaphoreType.DMA((2 * (N_DEV - 1),)),
            pltpu.SemaphoreType.DMA((2 * (N_DEV - 1),)),
        ],
        compiler_params=pltpu.CompilerParams(collective_id=0),
    )(A, B)


# baseline (device time: 393406 ns/iter reference)
import jax
import jax.numpy as jnp
from jax import lax
from jax.experimental import pallas as pl
from jax.experimental.pallas import tpu as pltpu

N_DEV = 4


def kernel(A, B):
    A = A.astype(jnp.bfloat16)
    B = B.astype(jnp.bfloat16)
    M, K = A.shape
    _, N = B.shape
    CH = M // N_DEV

    def body(a_ref, b_ref, out_ref, comm_ref, send_sems, recv_sems):
        p = lax.axis_index("i")
        left = (p - 1) % N_DEV
        right = (p + 1) % N_DEV

        barrier_sem = pltpu.get_barrier_semaphore()
        for nbr in [left, right]:
            pl.semaphore_signal(
                barrier_sem, inc=1,
                device_id=(nbr,), device_id_type=pl.DeviceIdType.MESH,
            )
        pl.semaphore_wait(barrier_sem, 2)

        for c in range(N_DEV):
            out_ref[pl.ds(c * CH, CH), :] = jnp.dot(
                a_ref[pl.ds(c * CH, CH), :], b_ref[:, :],
                preferred_element_type=jnp.float32,
            ).astype(jnp.bfloat16)

        for s in range(N_DEV - 1):
            send_c = (p + (N_DEV - 1) - s) % N_DEV
            recv_c = (p + (N_DEV - 2) - s) % N_DEV
            rdma = pltpu.make_async_remote_copy(
                src_ref=out_ref.at[pl.ds(send_c * CH, CH), :],
                dst_ref=comm_ref.at[s],
                send_sem=send_sems.at[s],
                recv_sem=recv_sems.at[s],
                device_id=(right,),
                device_id_type=pl.DeviceIdType.MESH,
            )
            rdma.start()
            rdma.wait()
            out_ref[pl.ds(recv_c * CH, CH), :] = (
                out_ref[pl.ds(recv_c * CH, CH), :] + comm_ref[s]
            )

        for s in range(N_DEV - 1):
            send_c = (p - s) % N_DEV
            rdma = pltpu.make_async_remote_copy(
                src_ref=out_ref.at[pl.ds(send_c * CH, CH), :],
                dst_ref=out_ref.at[pl.ds(send_c * CH, CH), :],
                send_sem=send_sems.at[N_DEV - 1 + s],
                recv_sem=recv_sems.at[N_DEV - 1 + s],
                device_id=(right,),
                device_id_type=pl.DeviceIdType.MESH,
            )
            rdma.start()
            rdma.wait()

    return pl.pallas_call(
        body,
        out_shape=jax.ShapeDtypeStruct((M, N), jnp.bfloat16),
        in_specs=[
            pl.BlockSpec(memory_space=pltpu.VMEM),
            pl.BlockSpec(memory_space=pltpu.VMEM),
        ],
        out_specs=pl.BlockSpec(memory_space=pltpu.VMEM),
        scratch_shapes=[
            pltpu.VMEM((N_DEV - 1, CH, N), jnp.bfloat16),
            pltpu.SemaphoreType.DMA((2 * (N_DEV - 1),)),
            pltpu.SemaphoreType.DMA((2 * (N_DEV - 1),)),
        ],
        compiler_params=pltpu.CompilerParams(
            collective_id=0,
            vmem_limit_bytes=100 * 1024 * 1024,
        ),
    )(A, B)


# device time: 241900 ns/iter; 1.6263x vs baseline; 1.6263x over previous
import jax
import jax.numpy as jnp
from jax import lax
from jax.experimental import pallas as pl
from jax.experimental.pallas import tpu as pltpu

N_DEV = 4


def kernel(A, B):
    A = A.astype(jnp.bfloat16)
    B = B.astype(jnp.bfloat16)
    M, K = A.shape
    _, N = B.shape
    CH = M // N_DEV
    HN = N // 2

    def body(a_ref, b_ref, out_ref, comm_r, comm_l,
             send_r, recv_r, send_l, recv_l):
        p = lax.axis_index("i")
        left = (p - 1) % N_DEV
        right = (p + 1) % N_DEV

        barrier_sem = pltpu.get_barrier_semaphore()
        for nbr in [left, right]:
            pl.semaphore_signal(
                barrier_sem, inc=1,
                device_id=(nbr,), device_id_type=pl.DeviceIdType.MESH,
            )
        pl.semaphore_wait(barrier_sem, 2)

        for c in range(N_DEV):
            out_ref[pl.ds(c * CH, CH), :] = jnp.dot(
                a_ref[pl.ds(c * CH, CH), :], b_ref[:, :],
                preferred_element_type=jnp.float32,
            ).astype(jnp.bfloat16)

        for s in range(N_DEV - 1):
            send_cr = (p + (N_DEV - 1) - s) % N_DEV
            recv_cr = (p + (N_DEV - 2) - s) % N_DEV
            send_cl = (p - (N_DEV - 1) + s) % N_DEV
            recv_cl = (p - (N_DEV - 2) + s) % N_DEV
            rdma_r = pltpu.make_async_remote_copy(
                src_ref=out_ref.at[pl.ds(send_cr * CH, CH), pl.ds(0, HN)],
                dst_ref=comm_r.at[s],
                send_sem=send_r.at[s],
                recv_sem=recv_r.at[s],
                device_id=(right,),
                device_id_type=pl.DeviceIdType.MESH,
            )
            rdma_l = pltpu.make_async_remote_copy(
                src_ref=out_ref.at[pl.ds(send_cl * CH, CH), pl.ds(HN, HN)],
                dst_ref=comm_l.at[s],
                send_sem=send_l.at[s],
                recv_sem=recv_l.at[s],
                device_id=(left,),
                device_id_type=pl.DeviceIdType.MESH,
            )
            rdma_r.start()
            rdma_l.start()
            rdma_r.wait()
            rdma_l.wait()
            out_ref[pl.ds(recv_cr * CH, CH), pl.ds(0, HN)] = (
                out_ref[pl.ds(recv_cr * CH, CH), pl.ds(0, HN)] + comm_r[s]
            )
            out_ref[pl.ds(recv_cl * CH, CH), pl.ds(HN, HN)] = (
                out_ref[pl.ds(recv_cl * CH, CH), pl.ds(HN, HN)] + comm_l[s]
            )

        for s in range(N_DEV - 1):
            send_cr = (p - s) % N_DEV
            send_cl = (p + s) % N_DEV
            rdma_r = pltpu.make_async_remote_copy(
                src_ref=out_ref.at[pl.ds(send_cr * CH, CH), pl.ds(0, HN)],
                dst_ref=out_ref.at[pl.ds(send_cr * CH, CH), pl.ds(0, HN)],
                send_sem=send_r.at[N_DEV - 1 + s],
                recv_sem=recv_r.at[N_DEV - 1 + s],
                device_id=(right,),
                device_id_type=pl.DeviceIdType.MESH,
            )
            rdma_l = pltpu.make_async_remote_copy(
                src_ref=out_ref.at[pl.ds(send_cl * CH, CH), pl.ds(HN, HN)],
                dst_ref=out_ref.at[pl.ds(send_cl * CH, CH), pl.ds(HN, HN)],
                send_sem=send_l.at[N_DEV - 1 + s],
                recv_sem=recv_l.at[N_DEV - 1 + s],
                device_id=(left,),
                device_id_type=pl.DeviceIdType.MESH,
            )
            rdma_r.start()
            rdma_l.start()
            rdma_r.wait()
            rdma_l.wait()

    n_sems = 2 * (N_DEV - 1)
    return pl.pallas_call(
        body,
        out_shape=jax.ShapeDtypeStruct((M, N), jnp.bfloat16),
        in_specs=[
            pl.BlockSpec(memory_space=pltpu.VMEM),
            pl.BlockSpec(memory_space=pltpu.VMEM),
        ],
        out_specs=pl.BlockSpec(memory_space=pltpu.VMEM),
        scratch_shapes=[
            pltpu.VMEM((N_DEV - 1, CH, HN), jnp.bfloat16),
            pltpu.VMEM((N_DEV - 1, CH, HN), jnp.bfloat16),
            pltpu.SemaphoreType.DMA((n_sems,)),
            pltpu.SemaphoreType.DMA((n_sems,)),
            pltpu.SemaphoreType.DMA((n_sems,)),
            pltpu.SemaphoreType.DMA((n_sems,)),
        ],
        compiler_params=pltpu.CompilerParams(
            collective_id=0,
            vmem_limit_bytes=100 * 1024 * 1024,
        ),
    )(A, B)


# device time: 219228 ns/iter; 1.7945x vs baseline; 1.1034x over previous
import jax
import jax.numpy as jnp
from jax import lax
from jax.experimental import pallas as pl
from jax.experimental.pallas import tpu as pltpu

N_DEV = 4


def kernel(A, B):
    A = A.astype(jnp.bfloat16)
    B = B.astype(jnp.bfloat16)
    M, K = A.shape
    _, N = B.shape
    CH = M // N_DEV
    HN = N // 2

    def body(a_ref, b_ref, out_ref, comm_r, comm_l,
             send_r, recv_r, send_l, recv_l):
        p = lax.axis_index("i")
        left = (p - 1) % N_DEV
        right = (p + 1) % N_DEV

        barrier_sem = pltpu.get_barrier_semaphore()
        for nbr in [left, right]:
            pl.semaphore_signal(
                barrier_sem, inc=1,
                device_id=(nbr,), device_id_type=pl.DeviceIdType.MESH,
            )
        pl.semaphore_wait(barrier_sem, 2)

        def compute_half(c, half):
            out_ref[pl.ds(c * CH, CH), pl.ds(half * HN, HN)] = jnp.dot(
                a_ref[pl.ds(c * CH, CH), :],
                b_ref[:, pl.ds(half * HN, HN)],
                preferred_element_type=jnp.float32,
            ).astype(jnp.bfloat16)

        compute_half((p + 3) % N_DEV, 0)
        compute_half((p + 1) % N_DEV, 1)

        for s in range(N_DEV - 1):
            send_cr = (p + (N_DEV - 1) - s) % N_DEV
            recv_cr = (p + (N_DEV - 2) - s) % N_DEV
            send_cl = (p - (N_DEV - 1) + s) % N_DEV
            recv_cl = (p - (N_DEV - 2) + s) % N_DEV
            rdma_r = pltpu.make_async_remote_copy(
                src_ref=out_ref.at[pl.ds(send_cr * CH, CH), pl.ds(0, HN)],
                dst_ref=comm_r.at[s],
                send_sem=send_r.at[s],
                recv_sem=recv_r.at[s],
                device_id=(right,),
                device_id_type=pl.DeviceIdType.MESH,
            )
            rdma_l = pltpu.make_async_remote_copy(
                src_ref=out_ref.at[pl.ds(send_cl * CH, CH), pl.ds(HN, HN)],
                dst_ref=comm_l.at[s],
                send_sem=send_l.at[s],
                recv_sem=recv_l.at[s],
                device_id=(left,),
                device_id_type=pl.DeviceIdType.MESH,
            )
            rdma_r.start()
            rdma_l.start()
            compute_half(recv_cr, 0)
            compute_half(recv_cl, 1)
            rdma_r.wait()
            rdma_l.wait()
            out_ref[pl.ds(recv_cr * CH, CH), pl.ds(0, HN)] = (
                out_ref[pl.ds(recv_cr * CH, CH), pl.ds(0, HN)] + comm_r[s]
            )
            out_ref[pl.ds(recv_cl * CH, CH), pl.ds(HN, HN)] = (
                out_ref[pl.ds(recv_cl * CH, CH), pl.ds(HN, HN)] + comm_l[s]
            )

        for s in range(N_DEV - 1):
            send_cr = (p - s) % N_DEV
            send_cl = (p + s) % N_DEV
            rdma_r = pltpu.make_async_remote_copy(
                src_ref=out_ref.at[pl.ds(send_cr * CH, CH), pl.ds(0, HN)],
                dst_ref=out_ref.at[pl.ds(send_cr * CH, CH), pl.ds(0, HN)],
                send_sem=send_r.at[N_DEV - 1 + s],
                recv_sem=recv_r.at[N_DEV - 1 + s],
                device_id=(right,),
                device_id_type=pl.DeviceIdType.MESH,
            )
            rdma_l = pltpu.make_async_remote_copy(
                src_ref=out_ref.at[pl.ds(send_cl * CH, CH), pl.ds(HN, HN)],
                dst_ref=out_ref.at[pl.ds(send_cl * CH, CH), pl.ds(HN, HN)],
                send_sem=send_l.at[N_DEV - 1 + s],
                recv_sem=recv_l.at[N_DEV - 1 + s],
                device_id=(left,),
                device_id_type=pl.DeviceIdType.MESH,
            )
            rdma_r.start()
            rdma_l.start()
            rdma_r.wait()
            rdma_l.wait()

    n_sems = 2 * (N_DEV - 1)
    return pl.pallas_call(
        body,
        out_shape=jax.ShapeDtypeStruct((M, N), jnp.bfloat16),
        in_specs=[
            pl.BlockSpec(memory_space=pltpu.VMEM),
            pl.BlockSpec(memory_space=pltpu.VMEM),
        ],
        out_specs=pl.BlockSpec(memory_space=pltpu.VMEM),
        scratch_shapes=[
            pltpu.VMEM((N_DEV - 1, CH, HN), jnp.bfloat16),
            pltpu.VMEM((N_DEV - 1, CH, HN), jnp.bfloat16),
            pltpu.SemaphoreType.DMA((n_sems,)),
            pltpu.SemaphoreType.DMA((n_sems,)),
            pltpu.SemaphoreType.DMA((n_sems,)),
            pltpu.SemaphoreType.DMA((n_sems,)),
        ],
        compiler_params=pltpu.CompilerParams(
            collective_id=0,
            vmem_limit_bytes=100 * 1024 * 1024,
        ),
    )(A, B)


# device time: 211342 ns/iter; 1.8615x vs baseline; 1.0373x over previous
import jax
import jax.numpy as jnp
from jax import lax
from jax.experimental import pallas as pl
from jax.experimental.pallas import tpu as pltpu

N_DEV = 4
N_SUB = 2


def kernel(A, B):
    A = A.astype(jnp.bfloat16)
    B = B.astype(jnp.bfloat16)
    M, K = A.shape
    _, N = B.shape
    CH = M // N_DEV
    HN = N // 2
    SUB = HN // N_SUB

    def body(a_ref, b_ref, out_ref, comm_r, comm_l,
             srs_r, rrs_r, srs_l, rrs_l,
             sag_r, rag_r, sag_l, rag_l):
        p = lax.axis_index("i")
        left = (p - 1) % N_DEV
        right = (p + 1) % N_DEV

        barrier_sem = pltpu.get_barrier_semaphore()
        for nbr in [left, right]:
            pl.semaphore_signal(
                barrier_sem, inc=1,
                device_id=(nbr,), device_id_type=pl.DeviceIdType.MESH,
            )
        pl.semaphore_wait(barrier_sem, 2)

        def compute_half(c, half):
            out_ref[pl.ds(c * CH, CH), pl.ds(half * HN, HN)] = jnp.dot(
                a_ref[pl.ds(c * CH, CH), :],
                b_ref[:, pl.ds(half * HN, HN)],
                preferred_element_type=jnp.float32,
            ).astype(jnp.bfloat16)

        def rs_send_row(ring, s):
            return ((p + (N_DEV - 1) - s) if ring == 0
                    else (p - (N_DEV - 1) + s)) % N_DEV

        def rs_recv_row(ring, s):
            return ((p + (N_DEV - 2) - s) if ring == 0
                    else (p - (N_DEV - 2) + s)) % N_DEV

        def col0(ring, h):
            return (0 if ring == 0 else HN) + h * SUB

        def rs_rdma(ring, s, h):
            comm = comm_r if ring == 0 else comm_l
            ssem = srs_r if ring == 0 else srs_l
            rsem = rrs_r if ring == 0 else rrs_l
            i = N_SUB * s + h
            rdma = pltpu.make_async_remote_copy(
                src_ref=out_ref.at[pl.ds(rs_send_row(ring, s) * CH, CH),
                                   pl.ds(col0(ring, h), SUB)],
                dst_ref=comm.at[i],
                send_sem=ssem.at[i],
                recv_sem=rsem.at[i],
                device_id=(right if ring == 0 else left,),
                device_id_type=pl.DeviceIdType.MESH,
            )
            rdma.start()
            return rdma

        def rs_add(ring, s, h):
            comm = comm_r if ring == 0 else comm_l
            rows = pl.ds(rs_recv_row(ring, s) * CH, CH)
            cols = pl.ds(col0(ring, h), SUB)
            out_ref[rows, cols] = out_ref[rows, cols] + comm[N_SUB * s + h]

        compute_half((p + (N_DEV - 1)) % N_DEV, 0)
        compute_half((p + 1) % N_DEV, 1)

        rs_d = {}
        for ring in range(2):
            for h in range(N_SUB):
                rs_d[(ring, 0, h)] = rs_rdma(ring, 0, h)
        for s in range(N_DEV - 1):
            compute_half(rs_recv_row(0, s), 0)
            compute_half(rs_recv_row(1, s), 1)
            for h in range(N_SUB):
                for ring in range(2):
                    rs_d[(ring, s, h)].wait()
                    rs_add(ring, s, h)
                    if s < N_DEV - 2:
                        rs_d[(ring, s + 1, h)] = rs_rdma(ring, s + 1, h)

        def ag_rdma(ring, s, h):
            row = ((p - s) if ring == 0 else (p + s)) % N_DEV
            src = out_ref.at[pl.ds(row * CH, CH), pl.ds(col0(ring, h), SUB)]
            ssem = sag_r if ring == 0 else sag_l
            rsem = rag_r if ring == 0 else rag_l
            i = N_SUB * s + h
            rdma = pltpu.make_async_remote_copy(
                src_ref=src,
                dst_ref=src,
                send_sem=ssem.at[i],
                recv_sem=rsem.at[i],
                device_id=(right if ring == 0 else left,),
                device_id_type=pl.DeviceIdType.MESH,
            )
            rdma.start()
            return rdma

        ag_d = {}
        for ring in range(2):
            for h in range(N_SUB):
                ag_d[(ring, 0, h)] = ag_rdma(ring, 0, h)
        for s in range(N_DEV - 1):
            for h in range(N_SUB):
                for ring in range(2):
                    ag_d[(ring, s, h)].wait()
                    if s < N_DEV - 2:
                        ag_d[(ring, s + 1, h)] = ag_rdma(ring, s + 1, h)

    n_sems = N_SUB * (N_DEV - 1)
    return pl.pallas_call(
        body,
        out_shape=jax.ShapeDtypeStruct((M, N), jnp.bfloat16),
        in_specs=[
            pl.BlockSpec(memory_space=pltpu.VMEM),
            pl.BlockSpec(memory_space=pltpu.VMEM),
        ],
        out_specs=pl.BlockSpec(memory_space=pltpu.VMEM),
        scratch_shapes=[
            pltpu.VMEM((n_sems, CH, SUB), jnp.bfloat16),
            pltpu.VMEM((n_sems, CH, SUB), jnp.bfloat16),
            pltpu.SemaphoreType.DMA((n_sems,)),
            pltpu.SemaphoreType.DMA((n_sems,)),
            pltpu.SemaphoreType.DMA((n_sems,)),
            pltpu.SemaphoreType.DMA((n_sems,)),
            pltpu.SemaphoreType.DMA((n_sems,)),
            pltpu.SemaphoreType.DMA((n_sems,)),
            pltpu.SemaphoreType.DMA((n_sems,)),
            pltpu.SemaphoreType.DMA((n_sems,)),
        ],
        compiler_params=pltpu.CompilerParams(
            collective_id=0,
            vmem_limit_bytes=100 * 1024 * 1024,
        ),
    )(A, B)


# device time: 193235 ns/iter; 2.0359x vs baseline; 1.0937x over previous
import jax
import jax.numpy as jnp
from jax import lax
from jax.experimental import pallas as pl
from jax.experimental.pallas import tpu as pltpu

N_DEV = 4
N_SUB = 2


def kernel(A, B):
    M, K = A.shape
    _, N = B.shape
    CH = M // N_DEV
    HCH = CH // 2
    HN = N // 2
    SUB = HN // N_SUB
    CB = N // 8
    NB = N // CB

    def body(a_hbm, b_hbm, out_ref, a_bf, b_bf, a_st, b_st,
             a_cp, b_cp, comm_r, comm_l,
             srs_r, rrs_r, srs_l, rrs_l,
             sag_r, rag_r, sag_l, rag_l):
        p = lax.axis_index("i")
        left = (p - 1) % N_DEV
        right = (p + 1) % N_DEV

        barrier_sem = pltpu.get_barrier_semaphore()
        for nbr in [left, right]:
            pl.semaphore_signal(
                barrier_sem, inc=1,
                device_id=(nbr,), device_id_type=pl.DeviceIdType.MESH,
            )
        pl.semaphore_wait(barrier_sem, 2)

        def a_copy(c, j, slot):
            cp = pltpu.make_async_copy(
                a_hbm.at[pl.ds(c * CH + j * HCH, HCH), :],
                a_st.at[slot], a_cp.at[slot],
            )
            cp.start()
            return (cp, c, j, slot)

        def a_fin(t):
            cp, c, j, slot = t
            cp.wait()
            a_bf[pl.ds(c * CH + j * HCH, HCH), :] = (
                a_st[slot].astype(jnp.bfloat16))

        def a_chunk(c):
            t0 = a_copy(c, 0, 0)
            t1 = a_copy(c, 1, 1)
            a_fin(t0)
            a_fin(t1)

        def b_copy(k, slot):
            cp = pltpu.make_async_copy(
                b_hbm.at[:, pl.ds(k * CB, CB)],
                b_st.at[slot], b_cp.at[slot],
            )
            cp.start()
            return (cp, k, slot)

        def b_fin(t):
            cp, k, slot = t
            cp.wait()
            b_bf[:, pl.ds(k * CB, CB)] = b_st[slot].astype(jnp.bfloat16)

        def b_pair(k0, k1):
            t0 = b_copy(k0, 0)
            t1 = b_copy(k1, 1)
            b_fin(t0)
            b_fin(t1)

        def compute_block(c, cb):
            out_ref[pl.ds(c * CH, CH), pl.ds(cb * CB, CB)] = jnp.dot(
                a_bf[pl.ds(c * CH, CH), :],
                b_bf[:, pl.ds(cb * CB, CB)],
                preferred_element_type=jnp.float32,
            ).astype(jnp.bfloat16)

        def rs_send_row(ring, s):
            return ((p + (N_DEV - 1) - s) if ring == 0
                    else (p - (N_DEV - 1) + s)) % N_DEV

        def rs_recv_row(ring, s):
            return ((p + (N_DEV - 2) - s) if ring == 0
                    else (p - (N_DEV - 2) + s)) % N_DEV

        def col0(ring, h):
            return (0 if ring == 0 else HN) + h * SUB

        def rs_rdma(ring, s, h):
            comm = comm_r if ring == 0 else comm_l
            ssem = srs_r if ring == 0 else srs_l
            rsem = rrs_r if ring == 0 else rrs_l
            i = N_SUB * s + h
            rdma = pltpu.make_async_remote_copy(
                src_ref=out_ref.at[pl.ds(rs_send_row(ring, s) * CH, CH),
                                   pl.ds(col0(ring, h), SUB)],
                dst_ref=comm.at[i],
                send_sem=ssem.at[i],
                recv_sem=rsem.at[i],
                device_id=(right if ring == 0 else left,),
                device_id_type=pl.DeviceIdType.MESH,
            )
            rdma.start()
            return rdma

        def rs_add(ring, s, h):
            comm = comm_r if ring == 0 else comm_l
            rows = pl.ds(rs_recv_row(ring, s) * CH, CH)
            cols = pl.ds(col0(ring, h), SUB)
            out_ref[rows, cols] = out_ref[rows, cols] + comm[N_SUB * s + h]

        rs_d = {}
        a_chunk((p + 3) % N_DEV)
        b_pair(0, 1)
        compute_block((p + 3) % N_DEV, 0)
        compute_block((p + 3) % N_DEV, 1)
        rs_d[(0, 0, 0)] = rs_rdma(0, 0, 0)

        a_chunk((p + 1) % N_DEV)
        b_pair(4, 5)
        compute_block((p + 1) % N_DEV, 4)
        compute_block((p + 1) % N_DEV, 5)
        rs_d[(1, 0, 0)] = rs_rdma(1, 0, 0)

        b_pair(2, 3)
        compute_block((p + 3) % N_DEV, 2)
        compute_block((p + 3) % N_DEV, 3)
        rs_d[(0, 0, 1)] = rs_rdma(0, 0, 1)

        b_pair(6, 7)
        compute_block((p + 1) % N_DEV, 6)
        compute_block((p + 1) % N_DEV, 7)
        rs_d[(1, 0, 1)] = rs_rdma(1, 0, 1)

        a_chunk((p + 2) % N_DEV)
        a_chunk(p)

        for s in range(N_DEV - 1):
            cr0 = rs_recv_row(0, s)
            cr1 = rs_recv_row(1, s)
            for cb in range(NB // 2):
                compute_block(cr0, cb)
            for cb in range(NB // 2, NB):
                compute_block(cr1, cb)
            for h in range(N_SUB):
                for ring in range(2):
                    rs_d[(ring, s, h)].wait()
                    rs_add(ring, s, h)
                    if s < N_DEV - 2:
                        rs_d[(ring, s + 1, h)] = rs_rdma(ring, s + 1, h)

        def ag_rdma(ring, s, h):
            row = ((p - s) if ring == 0 else (p + s)) % N_DEV
            src = out_ref.at[pl.ds(row * CH, CH), pl.ds(col0(ring, h), SUB)]
            ssem = sag_r if ring == 0 else sag_l
            rsem = rag_r if ring == 0 else rag_l
            i = N_SUB * s + h
            rdma = pltpu.make_async_remote_copy(
                src_ref=src,
                dst_ref=src,
                send_sem=ssem.at[i],
                recv_sem=rsem.at[i],
                device_id=(right if ring == 0 else left,),
                device_id_type=pl.DeviceIdType.MESH,
            )
            rdma.start()
            return rdma

        ag_d = {}
        for ring in range(2):
            for h in range(N_SUB):
                ag_d[(ring, 0, h)] = ag_rdma(ring, 0, h)
        for s in range(N_DEV - 1):
            for h in range(N_SUB):
                for ring in range(2):
                    ag_d[(ring, s, h)].wait()
                    if s < N_DEV - 2:
                        ag_d[(ring, s + 1, h)] = ag_rdma(ring, s + 1, h)

    n_sems = N_SUB * (N_DEV - 1)
    return pl.pallas_call(
        body,
        out_shape=jax.ShapeDtypeStruct((M, N), jnp.bfloat16),
        in_specs=[
            pl.BlockSpec(memory_space=pl.MemorySpace.ANY),
            pl.BlockSpec(memory_space=pl.MemorySpace.ANY),
        ],
        out_specs=pl.BlockSpec(memory_space=pltpu.VMEM),
        scratch_shapes=[
            pltpu.VMEM((M, K), jnp.bfloat16),
            pltpu.VMEM((K, N), jnp.bfloat16),
            pltpu.VMEM((2, HCH, K), jnp.float32),
            pltpu.VMEM((2, K, CB), jnp.float32),
            pltpu.SemaphoreType.DMA((2,)),
            pltpu.SemaphoreType.DMA((2,)),
            pltpu.VMEM((n_sems, CH, SUB), jnp.bfloat16),
            pltpu.VMEM((n_sems, CH, SUB), jnp.bfloat16),
            pltpu.SemaphoreType.DMA((n_sems,)),
            pltpu.SemaphoreType.DMA((n_sems,)),
            pltpu.SemaphoreType.DMA((n_sems,)),
            pltpu.SemaphoreType.DMA((n_sems,)),
            pltpu.SemaphoreType.DMA((n_sems,)),
            pltpu.SemaphoreType.DMA((n_sems,)),
            pltpu.SemaphoreType.DMA((n_sems,)),
            pltpu.SemaphoreType.DMA((n_sems,)),
        ],
        compiler_params=pltpu.CompilerParams(
            collective_id=0,
            vmem_limit_bytes=100 * 1024 * 1024,
        ),
    )(A, B)


# device time: 190697 ns/iter; 2.0630x vs baseline; 1.0133x over previous
import jax
import jax.numpy as jnp
from jax import lax
from jax.experimental import pallas as pl
from jax.experimental.pallas import tpu as pltpu

N_DEV = 4
N_SUB = 2


def kernel(A, B):
    M, K = A.shape
    _, N = B.shape
    CH = M // N_DEV
    HCH = CH // 2
    HN = N // 2
    SUB = HN // N_SUB
    CB = N // 8
    NB = N // CB

    def body(a_hbm, b_hbm, out_ref, a_bf, b_bf, a_st, b_st,
             a_cp, b_cp, comm_r, comm_l,
             srs_r, rrs_r, srs_l, rrs_l,
             sag_r, rag_r, sag_l, rag_l):
        p = lax.axis_index("i")
        left = (p - 1) % N_DEV
        right = (p + 1) % N_DEV

        barrier_sem = pltpu.get_barrier_semaphore()
        for nbr in [left, right]:
            pl.semaphore_signal(
                barrier_sem, inc=1,
                device_id=(nbr,), device_id_type=pl.DeviceIdType.MESH,
            )

        def a_copy(c, j, slot):
            cp = pltpu.make_async_copy(
                a_hbm.at[pl.ds(c * CH + j * HCH, HCH), :],
                a_st.at[slot], a_cp.at[slot],
            )
            cp.start()
            return (cp, c, j, slot)

        def a_fin(t):
            cp, c, j, slot = t
            cp.wait()
            a_bf[pl.ds(c * CH + j * HCH, HCH), :] = (
                a_st[slot].astype(jnp.bfloat16))

        def a_chunk(c):
            t0 = a_copy(c, 0, 0)
            t1 = a_copy(c, 1, 1)
            a_fin(t0)
            a_fin(t1)

        def b_copy(k, slot):
            cp = pltpu.make_async_copy(
                b_hbm.at[:, pl.ds(k * CB, CB)],
                b_st.at[slot], b_cp.at[slot],
            )
            cp.start()
            return (cp, k, slot)

        def b_fin(t):
            cp, k, slot = t
            cp.wait()
            b_bf[:, pl.ds(k * CB, CB)] = b_st[slot].astype(jnp.bfloat16)

        def b_pair(k0, k1):
            t0 = b_copy(k0, 0)
            t1 = b_copy(k1, 1)
            b_fin(t0)
            b_fin(t1)

        def compute_block(c, cb):
            out_ref[pl.ds(c * CH, CH), pl.ds(cb * CB, CB)] = jnp.dot(
                a_bf[pl.ds(c * CH, CH), :],
                b_bf[:, pl.ds(cb * CB, CB)],
                preferred_element_type=jnp.float32,
            ).astype(jnp.bfloat16)

        def rs_send_row(ring, s):
            return ((p + (N_DEV - 1) - s) if ring == 0
                    else (p - (N_DEV - 1) + s)) % N_DEV

        def rs_recv_row(ring, s):
            return ((p + (N_DEV - 2) - s) if ring == 0
                    else (p - (N_DEV - 2) + s)) % N_DEV

        def col0(ring, h):
            return (0 if ring == 0 else HN) + h * SUB

        def rs_rdma(ring, s, h):
            comm = comm_r if ring == 0 else comm_l
            ssem = srs_r if ring == 0 else srs_l
            rsem = rrs_r if ring == 0 else rrs_l
            i = N_SUB * s + h
            rdma = pltpu.make_async_remote_copy(
                src_ref=out_ref.at[pl.ds(rs_send_row(ring, s) * CH, CH),
                                   pl.ds(col0(ring, h), SUB)],
                dst_ref=comm.at[i],
                send_sem=ssem.at[i],
                recv_sem=rsem.at[i],
                device_id=(right if ring == 0 else left,),
                device_id_type=pl.DeviceIdType.MESH,
            )
            rdma.start()
            return rdma

        def rs_add(ring, s, h):
            comm = comm_r if ring == 0 else comm_l
            rows = pl.ds(rs_recv_row(ring, s) * CH, CH)
            cols = pl.ds(col0(ring, h), SUB)
            out_ref[rows, cols] = out_ref[rows, cols] + comm[N_SUB * s + h]

        rs_d = {}
        ta = [a_copy((p + 3) % N_DEV, 0, 0), a_copy((p + 3) % N_DEV, 1, 1)]
        tb = [b_copy(0, 0), b_copy(1, 1)]
        for t in ta:
            a_fin(t)
        for t in tb:
            b_fin(t)
        ta = [a_copy((p + 1) % N_DEV, 0, 0), a_copy((p + 1) % N_DEV, 1, 1)]
        tb = [b_copy(4, 0), b_copy(5, 1)]
        compute_block((p + 3) % N_DEV, 0)
        compute_block((p + 3) % N_DEV, 1)
        pl.semaphore_wait(barrier_sem, 2)
        rs_d[(0, 0, 0)] = rs_rdma(0, 0, 0)

        for t in ta:
            a_fin(t)
        for t in tb:
            b_fin(t)
        tb = [b_copy(2, 0), b_copy(3, 1)]
        compute_block((p + 1) % N_DEV, 4)
        compute_block((p + 1) % N_DEV, 5)
        rs_d[(1, 0, 0)] = rs_rdma(1, 0, 0)

        for t in tb:
            b_fin(t)
        tb = [b_copy(6, 0), b_copy(7, 1)]
        compute_block((p + 3) % N_DEV, 2)
        compute_block((p + 3) % N_DEV, 3)
        rs_d[(0, 0, 1)] = rs_rdma(0, 0, 1)

        for t in tb:
            b_fin(t)
        compute_block((p + 1) % N_DEV, 6)
        compute_block((p + 1) % N_DEV, 7)
        rs_d[(1, 0, 1)] = rs_rdma(1, 0, 1)

        a_chunk((p + 2) % N_DEV)
        a_chunk(p)

        for s in range(N_DEV - 1):
            cr0 = rs_recv_row(0, s)
            cr1 = rs_recv_row(1, s)
            for cb in range(NB // 2):
                compute_block(cr0, cb)
            for cb in range(NB // 2, NB):
                compute_block(cr1, cb)
            for h in range(N_SUB):
                for ring in range(2):
                    rs_d[(ring, s, h)].wait()
                    rs_add(ring, s, h)
                    if s < N_DEV - 2:
                        rs_d[(ring, s + 1, h)] = rs_rdma(ring, s + 1, h)

        def ag_rdma(ring, s, h):
            row = ((p - s) if ring == 0 else (p + s)) % N_DEV
            src = out_ref.at[pl.ds(row * CH, CH), pl.ds(col0(ring, h), SUB)]
            ssem = sag_r if ring == 0 else sag_l
            rsem = rag_r if ring == 0 else rag_l
            i = N_SUB * s + h
            rdma = pltpu.make_async_remote_copy(
                src_ref=src,
                dst_ref=src,
                send_sem=ssem.at[i],
                recv_sem=rsem.at[i],
                device_id=(right if ring == 0 else left,),
                device_id_type=pl.DeviceIdType.MESH,
            )
            rdma.start()
            return rdma

        ag_d = {}
        for ring in range(2):
            for h in range(N_SUB):
                ag_d[(ring, 0, h)] = ag_rdma(ring, 0, h)
        for s in range(N_DEV - 1):
            for h in range(N_SUB):
                for ring in range(2):
                    ag_d[(ring, s, h)].wait()
                    if s < N_DEV - 2:
                        ag_d[(ring, s + 1, h)] = ag_rdma(ring, s + 1, h)

    n_sems = N_SUB * (N_DEV - 1)
    return pl.pallas_call(
        body,
        out_shape=jax.ShapeDtypeStruct((M, N), jnp.bfloat16),
        in_specs=[
            pl.BlockSpec(memory_space=pl.MemorySpace.ANY),
            pl.BlockSpec(memory_space=pl.MemorySpace.ANY),
        ],
        out_specs=pl.BlockSpec(memory_space=pltpu.VMEM),
        scratch_shapes=[
            pltpu.VMEM((M, K), jnp.bfloat16),
            pltpu.VMEM((K, N), jnp.bfloat16),
            pltpu.VMEM((2, HCH, K), jnp.float32),
            pltpu.VMEM((2, K, CB), jnp.float32),
            pltpu.SemaphoreType.DMA((2,)),
            pltpu.SemaphoreType.DMA((2,)),
            pltpu.VMEM((n_sems, CH, SUB), jnp.bfloat16),
            pltpu.VMEM((n_sems, CH, SUB), jnp.bfloat16),
            pltpu.SemaphoreType.DMA((n_sems,)),
            pltpu.SemaphoreType.DMA((n_sems,)),
            pltpu.SemaphoreType.DMA((n_sems,)),
            pltpu.SemaphoreType.DMA((n_sems,)),
            pltpu.SemaphoreType.DMA((n_sems,)),
            pltpu.SemaphoreType.DMA((n_sems,)),
            pltpu.SemaphoreType.DMA((n_sems,)),
            pltpu.SemaphoreType.DMA((n_sems,)),
        ],
        compiler_params=pltpu.CompilerParams(
            collective_id=0,
            vmem_limit_bytes=100 * 1024 * 1024,
        ),
    )(A, B)


# device time: 188872 ns/iter; 2.0829x vs baseline; 1.0097x over previous
import os

import jax
import jax.numpy as jnp
from jax import lax
from jax.experimental import pallas as pl
from jax.experimental.pallas import tpu as pltpu

N_DEV = 4
N_SUB = 4

_COMM_ONLY = bool(os.environ.get("K_COMM_ONLY"))
_SKIP_AG = bool(os.environ.get("K_SKIP_AG"))


def kernel(A, B):
    M, K = A.shape
    _, N = B.shape
    CH = M // N_DEV
    HCH = CH // 2
    HN = N // 2
    SUB = HN // N_SUB
    CB = N // 8
    NB = N // CB

    def body(a_hbm, b_hbm, out_ref, a_bf, b_bf, a_st, b_st,
             a_cp, b_cp, comm_r, comm_l,
             srs_r, rrs_r, srs_l, rrs_l,
             sag_r, rag_r, sag_l, rag_l):
        p = lax.axis_index("i")
        left = (p - 1) % N_DEV
        right = (p + 1) % N_DEV

        barrier_sem = pltpu.get_barrier_semaphore()
        for nbr in [left, right]:
            pl.semaphore_signal(
                barrier_sem, inc=1,
                device_id=(nbr,), device_id_type=pl.DeviceIdType.MESH,
            )

        def a_copy(c, j, slot):
            cp = pltpu.make_async_copy(
                a_hbm.at[pl.ds(c * CH + j * HCH, HCH), :],
                a_st.at[slot], a_cp.at[slot],
            )
            cp.start()
            return (cp, c, j, slot)

        def a_fin(t):
            cp, c, j, slot = t
            cp.wait()
            a_bf[pl.ds(c * CH + j * HCH, HCH), :] = (
                a_st[slot].astype(jnp.bfloat16))

        def a_chunk(c):
            if _COMM_ONLY:
                return
            t0 = a_copy(c, 0, 0)
            t1 = a_copy(c, 1, 1)
            a_fin(t0)
            a_fin(t1)

        def b_copy(k, slot):
            cp = pltpu.make_async_copy(
                b_hbm.at[:, pl.ds(k * CB, CB)],
                b_st.at[slot], b_cp.at[slot],
            )
            cp.start()
            return (cp, k, slot)

        def b_fin(t):
            cp, k, slot = t
            cp.wait()
            b_bf[:, pl.ds(k * CB, CB)] = b_st[slot].astype(jnp.bfloat16)

        def b_pair(k0, k1):
            t0 = b_copy(k0, 0)
            t1 = b_copy(k1, 1)
            b_fin(t0)
            b_fin(t1)

        def compute_block(c, cb):
            if _COMM_ONLY:
                return
            out_ref[pl.ds(c * CH, CH), pl.ds(cb * CB, CB)] = jnp.dot(
                a_bf[pl.ds(c * CH, CH), :],
                b_bf[:, pl.ds(cb * CB, CB)],
                preferred_element_type=jnp.float32,
            ).astype(jnp.bfloat16)

        def rs_send_row(ring, s):
            return ((p + (N_DEV - 1) - s) if ring == 0
                    else (p - (N_DEV - 1) + s)) % N_DEV

        def rs_recv_row(ring, s):
            return ((p + (N_DEV - 2) - s) if ring == 0
                    else (p - (N_DEV - 2) + s)) % N_DEV

        def col0(ring, h):
            return (0 if ring == 0 else HN) + h * SUB

        def rs_rdma(ring, s, h):
            comm = comm_r if ring == 0 else comm_l
            ssem = srs_r if ring == 0 else srs_l
            rsem = rrs_r if ring == 0 else rrs_l
            i = N_SUB * s + h
            rdma = pltpu.make_async_remote_copy(
                src_ref=out_ref.at[pl.ds(rs_send_row(ring, s) * CH, CH),
                                   pl.ds(col0(ring, h), SUB)],
                dst_ref=comm.at[i],
                send_sem=ssem.at[i],
                recv_sem=rsem.at[i],
                device_id=(right if ring == 0 else left,),
                device_id_type=pl.DeviceIdType.MESH,
            )
            rdma.start()
            return rdma

        def rs_add(ring, s, h):
            comm = comm_r if ring == 0 else comm_l
            rows = pl.ds(rs_recv_row(ring, s) * CH, CH)
            cols = pl.ds(col0(ring, h), SUB)
            out_ref[rows, cols] = out_ref[rows, cols] + comm[N_SUB * s + h]

        rs_d = {}
        if not _COMM_ONLY:
            ta = [a_copy((p + 3) % N_DEV, 0, 0), a_copy((p + 3) % N_DEV, 1, 1)]
            tb = [b_copy(0, 0), b_copy(1, 1)]
            for t in ta:
                a_fin(t)
            for t in tb:
                b_fin(t)
            ta = [a_copy((p + 1) % N_DEV, 0, 0), a_copy((p + 1) % N_DEV, 1, 1)]
            tb = [b_copy(4, 0), b_copy(5, 1)]
        compute_block((p + 3) % N_DEV, 0)
        compute_block((p + 3) % N_DEV, 1)
        pl.semaphore_wait(barrier_sem, 2)
        rs_d[(0, 0, 0)] = rs_rdma(0, 0, 0)
        rs_d[(0, 0, 1)] = rs_rdma(0, 0, 1)

        if not _COMM_ONLY:
            for t in ta:
                a_fin(t)
            for t in tb:
                b_fin(t)
            tb = [b_copy(2, 0), b_copy(3, 1)]
        compute_block((p + 1) % N_DEV, 4)
        compute_block((p + 1) % N_DEV, 5)
        rs_d[(1, 0, 0)] = rs_rdma(1, 0, 0)
        rs_d[(1, 0, 1)] = rs_rdma(1, 0, 1)

        if not _COMM_ONLY:
            for t in tb:
                b_fin(t)
            tb = [b_copy(6, 0), b_copy(7, 1)]
        compute_block((p + 3) % N_DEV, 2)
        compute_block((p + 3) % N_DEV, 3)
        rs_d[(0, 0, 2)] = rs_rdma(0, 0, 2)
        rs_d[(0, 0, 3)] = rs_rdma(0, 0, 3)

        if not _COMM_ONLY:
            for t in tb:
                b_fin(t)
        compute_block((p + 1) % N_DEV, 6)
        compute_block((p + 1) % N_DEV, 7)
        rs_d[(1, 0, 2)] = rs_rdma(1, 0, 2)
        rs_d[(1, 0, 3)] = rs_rdma(1, 0, 3)

        a_chunk((p + 2) % N_DEV)
        a_chunk(p)

        def ag_rdma(ring, s, h):
            row = ((p - s) if ring == 0 else (p + s)) % N_DEV
            src = out_ref.at[pl.ds(row * CH, CH), pl.ds(col0(ring, h), SUB)]
            ssem = sag_r if ring == 0 else sag_l
            rsem = rag_r if ring == 0 else rag_l
            i = N_SUB * s + h
            rdma = pltpu.make_async_remote_copy(
                src_ref=src,
                dst_ref=src,
                send_sem=ssem.at[i],
                recv_sem=rsem.at[i],
                device_id=(right if ring == 0 else left,),
                device_id_type=pl.DeviceIdType.MESH,
            )
            rdma.start()
            return rdma

        ag_d = {}
        for s in range(N_DEV - 1):
            cr0 = rs_recv_row(0, s)
            cr1 = rs_recv_row(1, s)
            for cb in range(NB // 2):
                compute_block(cr0, cb)
            for cb in range(NB // 2, NB):
                compute_block(cr1, cb)
            for h in range(N_SUB):
                for ring in range(2):
                    rs_d[(ring, s, h)].wait()
                    rs_add(ring, s, h)
                    if s < N_DEV - 2:
                        rs_d[(ring, s + 1, h)] = rs_rdma(ring, s + 1, h)
                    elif not _SKIP_AG:
                        ag_d[(ring, 0, h)] = ag_rdma(ring, 0, h)

        if not _SKIP_AG:
            for s in range(N_DEV - 1):
                for h in range(N_SUB):
                    for ring in range(2):
                        ag_d[(ring, s, h)].wait()
                        if s < N_DEV - 2:
                            ag_d[(ring, s + 1, h)] = ag_rdma(ring, s + 1, h)

    n_sems = N_SUB * (N_DEV - 1)
    return pl.pallas_call(
        body,
        out_shape=jax.ShapeDtypeStruct((M, N), jnp.bfloat16),
        in_specs=[
            pl.BlockSpec(memory_space=pl.MemorySpace.ANY),
            pl.BlockSpec(memory_space=pl.MemorySpace.ANY),
        ],
        out_specs=pl.BlockSpec(memory_space=pltpu.VMEM),
        scratch_shapes=[
            pltpu.VMEM((M, K), jnp.bfloat16),
            pltpu.VMEM((K, N), jnp.bfloat16),
            pltpu.VMEM((2, HCH, K), jnp.float32),
            pltpu.VMEM((2, K, CB), jnp.float32),
            pltpu.SemaphoreType.DMA((2,)),
            pltpu.SemaphoreType.DMA((2,)),
            pltpu.VMEM((n_sems, CH, SUB), jnp.bfloat16),
            pltpu.VMEM((n_sems, CH, SUB), jnp.bfloat16),
            pltpu.SemaphoreType.DMA((n_sems,)),
            pltpu.SemaphoreType.DMA((n_sems,)),
            pltpu.SemaphoreType.DMA((n_sems,)),
            pltpu.SemaphoreType.DMA((n_sems,)),
            pltpu.SemaphoreType.DMA((n_sems,)),
            pltpu.SemaphoreType.DMA((n_sems,)),
            pltpu.SemaphoreType.DMA((n_sems,)),
            pltpu.SemaphoreType.DMA((n_sems,)),
        ],
        compiler_params=pltpu.CompilerParams(
            collective_id=0,
            vmem_limit_bytes=100 * 1024 * 1024,
        ),
    )(A, B)


# device time: 188057 ns/iter; 2.0920x vs baseline; 1.0043x over previous
import os

import jax
import jax.numpy as jnp
from jax import lax
from jax.experimental import pallas as pl
from jax.experimental.pallas import tpu as pltpu

N_DEV = 4
N_SUB = 4

_COMM_ONLY = bool(os.environ.get("K_COMM_ONLY"))
_SKIP_AG = bool(os.environ.get("K_SKIP_AG"))


def kernel(A, B):
    M, K = A.shape
    _, N = B.shape
    CH = M // N_DEV
    HCH = CH // 2
    HN = N // 2
    SUB = HN // N_SUB
    CB = N // 8
    NB = N // CB

    def body(a_hbm, b_hbm, out_ref, a_bf, b_bf, a_st, b_st,
             a_cp, b_cp, comm_r, comm_l,
             srs_r, rrs_r, srs_l, rrs_l,
             sag_r, rag_r, sag_l, rag_l):
        p = lax.axis_index("i")
        left = (p - 1) % N_DEV
        right = (p + 1) % N_DEV

        barrier_sem = pltpu.get_barrier_semaphore()
        for nbr in [left, right]:
            pl.semaphore_signal(
                barrier_sem, inc=1,
                device_id=(nbr,), device_id_type=pl.DeviceIdType.MESH,
            )

        def a_copy(c, j, slot):
            cp = pltpu.make_async_copy(
                a_hbm.at[pl.ds(c * CH + j * HCH, HCH), :],
                a_st.at[slot], a_cp.at[slot],
            )
            cp.start()
            return (cp, c, j, slot)

        def a_fin(t):
            cp, c, j, slot = t
            cp.wait()
            a_bf[pl.ds(c * CH + j * HCH, HCH), :] = (
                a_st[slot].astype(jnp.bfloat16))

        def a_chunk(c):
            if _COMM_ONLY:
                return
            t0 = a_copy(c, 0, 0)
            t1 = a_copy(c, 1, 1)
            a_fin(t0)
            a_fin(t1)

        def b_copy(k, slot):
            cp = pltpu.make_async_copy(
                b_hbm.at[:, pl.ds(k * CB, CB)],
                b_st.at[slot], b_cp.at[slot],
            )
            cp.start()
            return (cp, k, slot)

        def b_fin(t):
            cp, k, slot = t
            cp.wait()
            b_bf[:, pl.ds(k * CB, CB)] = b_st[slot].astype(jnp.bfloat16)

        def b_pair(k0, k1):
            t0 = b_copy(k0, 0)
            t1 = b_copy(k1, 1)
            b_fin(t0)
            b_fin(t1)

        def compute_block(c, cb):
            if _COMM_ONLY:
                return
            out_ref[pl.ds(c * CH, CH), pl.ds(cb * CB, CB)] = jnp.dot(
                a_bf[pl.ds(c * CH, CH), :],
                b_bf[:, pl.ds(cb * CB, CB)],
                preferred_element_type=jnp.float32,
            ).astype(jnp.bfloat16)

        def rs_send_row(ring, s):
            return ((p + (N_DEV - 1) - s) if ring == 0
                    else (p - (N_DEV - 1) + s)) % N_DEV

        def rs_recv_row(ring, s):
            return ((p + (N_DEV - 2) - s) if ring == 0
                    else (p - (N_DEV - 2) + s)) % N_DEV

        def col0(ring, h):
            return (0 if ring == 0 else HN) + h * SUB

        def rs_rdma(ring, s, h):
            comm = comm_r if ring == 0 else comm_l
            ssem = srs_r if ring == 0 else srs_l
            rsem = rrs_r if ring == 0 else rrs_l
            i = N_SUB * s + h
            rdma = pltpu.make_async_remote_copy(
                src_ref=out_ref.at[pl.ds(rs_send_row(ring, s) * CH, CH),
                                   pl.ds(col0(ring, h), SUB)],
                dst_ref=comm.at[i],
                send_sem=ssem.at[i],
                recv_sem=rsem.at[i],
                device_id=(right if ring == 0 else left,),
                device_id_type=pl.DeviceIdType.MESH,
            )
            rdma.start()
            return rdma

        def rs_add(ring, s, h):
            comm = comm_r if ring == 0 else comm_l
            rows = pl.ds(rs_recv_row(ring, s) * CH, CH)
            cols = pl.ds(col0(ring, h), SUB)
            out_ref[rows, cols] = out_ref[rows, cols] + comm[N_SUB * s + h]

        rs_d = {}
        if not _COMM_ONLY:
            ta = [a_copy((p + 3) % N_DEV, 0, 0), a_copy((p + 3) % N_DEV, 1, 1)]
            tb = [b_copy(0, 0), b_copy(1, 1)]
            for t in ta:
                a_fin(t)
            for t in tb:
                b_fin(t)
            ta = [a_copy((p + 1) % N_DEV, 0, 0), a_copy((p + 1) % N_DEV, 1, 1)]
            tb = [b_copy(4, 0), b_copy(5, 1)]
        compute_block((p + 3) % N_DEV, 0)
        compute_block((p + 3) % N_DEV, 1)
        pl.semaphore_wait(barrier_sem, 2)
        rs_d[(0, 0, 0)] = rs_rdma(0, 0, 0)
        rs_d[(0, 0, 1)] = rs_rdma(0, 0, 1)

        if not _COMM_ONLY:
            for t in ta:
                a_fin(t)
            for t in tb:
                b_fin(t)
            tb = [b_copy(2, 0), b_copy(3, 1)]
        compute_block((p + 1) % N_DEV, 4)
        compute_block((p + 1) % N_DEV, 5)
        rs_d[(1, 0, 0)] = rs_rdma(1, 0, 0)
        rs_d[(1, 0, 1)] = rs_rdma(1, 0, 1)

        if not _COMM_ONLY:
            for t in tb:
                b_fin(t)
            tb = [b_copy(6, 0), b_copy(7, 1)]
        compute_block((p + 3) % N_DEV, 2)
        compute_block((p + 3) % N_DEV, 3)
        rs_d[(0, 0, 2)] = rs_rdma(0, 0, 2)
        rs_d[(0, 0, 3)] = rs_rdma(0, 0, 3)

        if not _COMM_ONLY:
            for t in tb:
                b_fin(t)
        compute_block((p + 1) % N_DEV, 6)
        compute_block((p + 1) % N_DEV, 7)
        rs_d[(1, 0, 2)] = rs_rdma(1, 0, 2)
        rs_d[(1, 0, 3)] = rs_rdma(1, 0, 3)

        a_chunk((p + 2) % N_DEV)
        a_chunk(p)

        def ag_rdma(ring, s, h):
            row = ((p - s) if ring == 0 else (p + s)) % N_DEV
            src = out_ref.at[pl.ds(row * CH, CH), pl.ds(col0(ring, h), SUB)]
            ssem = sag_r if ring == 0 else sag_l
            rsem = rag_r if ring == 0 else rag_l
            i = N_SUB * s + h
            rdma = pltpu.make_async_remote_copy(
                src_ref=src,
                dst_ref=src,
                send_sem=ssem.at[i],
                recv_sem=rsem.at[i],
                device_id=(right if ring == 0 else left,),
                device_id_type=pl.DeviceIdType.MESH,
            )
            rdma.start()
            return rdma

        ag_d = {}
        for s in range(N_DEV - 1):
            cr0 = rs_recv_row(0, s)
            cr1 = rs_recv_row(1, s)
            for h in range(N_SUB):
                compute_block(cr0, h)
                compute_block(cr1, NB // 2 + h)
                for ring in range(2):
                    rs_d[(ring, s, h)].wait()
                    rs_add(ring, s, h)
                    if s < N_DEV - 2:
                        rs_d[(ring, s + 1, h)] = rs_rdma(ring, s + 1, h)
                    elif not _SKIP_AG:
                        ag_d[(ring, 0, h)] = ag_rdma(ring, 0, h)

        if not _SKIP_AG:
            for s in range(N_DEV - 1):
                for h in range(N_SUB):
                    for ring in range(2):
                        ag_d[(ring, s, h)].wait()
                        if s < N_DEV - 2:
                            ag_d[(ring, s + 1, h)] = ag_rdma(ring, s + 1, h)

    n_sems = N_SUB * (N_DEV - 1)
    return pl.pallas_call(
        body,
        out_shape=jax.ShapeDtypeStruct((M, N), jnp.bfloat16),
        in_specs=[
            pl.BlockSpec(memory_space=pl.MemorySpace.ANY),
            pl.BlockSpec(memory_space=pl.MemorySpace.ANY),
        ],
        out_specs=pl.BlockSpec(memory_space=pltpu.VMEM),
        scratch_shapes=[
            pltpu.VMEM((M, K), jnp.bfloat16),
            pltpu.VMEM((K, N), jnp.bfloat16),
            pltpu.VMEM((2, HCH, K), jnp.float32),
            pltpu.VMEM((2, K, CB), jnp.float32),
            pltpu.SemaphoreType.DMA((2,)),
            pltpu.SemaphoreType.DMA((2,)),
            pltpu.VMEM((n_sems, CH, SUB), jnp.bfloat16),
            pltpu.VMEM((n_sems, CH, SUB), jnp.bfloat16),
            pltpu.SemaphoreType.DMA((n_sems,)),
            pltpu.SemaphoreType.DMA((n_sems,)),
            pltpu.SemaphoreType.DMA((n_sems,)),
            pltpu.SemaphoreType.DMA((n_sems,)),
            pltpu.SemaphoreType.DMA((n_sems,)),
            pltpu.SemaphoreType.DMA((n_sems,)),
            pltpu.SemaphoreType.DMA((n_sems,)),
            pltpu.SemaphoreType.DMA((n_sems,)),
        ],
        compiler_params=pltpu.CompilerParams(
            collective_id=0,
            vmem_limit_bytes=100 * 1024 * 1024,
        ),
    )(A, B)
